# baseline (speedup 1.0000x reference)
import functools

import jax
import jax.numpy as jnp
from jax import lax
from jax.experimental import pallas as pl
from jax.experimental.pallas import tpu as pltpu
from jax.experimental.pallas import tpu_sc as plsc

_NC = 2
_NS = 16
_LANES = 16
_CHUNK = 80
_PASS_CHUNKS = 16
_NBUF = 1
_AHEAD = 1


def _n_pad_rows(n):
    align = _NS * 8
    return ((n + align - 1) // align) * align


def _sc_body(n_passes, rows_per_tile, nvec,
             x_hbm, src_hbm, dst_hbm, adj_hbm, partial_hbm,
             src_all, dst_all, adj_all, bufs, acc, gsems, ssems):
    c = lax.axis_index("c")
    s = lax.axis_index("s")
    wid = c * _NS + s

    zero = jnp.zeros((_LANES,), jnp.float32)
    for r in range(8):
        for k in range(nvec):
            bufs[0][r, pl.ds(k * _LANES, _LANES)] = zero

    def zcopy(b, carry):
        pltpu.sync_copy(bufs[0].at[pl.ds(0, 8)],
                        acc.at[pl.ds(s * rows_per_tile + b * 8, 8)])
        return carry

    lax.fori_loop(0, rows_per_tile // 8, zcopy, 0)
    plsc.subcore_barrier()

    chunk_base = wid * (n_passes * _PASS_CHUNKS)

    def wait_gather(buf, sem):
        pltpu.make_async_copy(x_hbm.at[pl.ds(0, _CHUNK)], buf, sem).wait()

    def wait_scatter(buf, idx_row, sem):
        pltpu.make_async_copy(buf, acc.at[dst_all.at[idx_row]], sem).wait()

    def compute_scale(buf, ci):
        def scale_rows(base_row, a16, count):
            for j in range(count):
                av = jnp.full((_LANES,), a16[j], jnp.float32)
                row = base_row + j
                for k in range(nvec):
                    sl = pl.ds(k * _LANES, _LANES)
                    buf[row, sl] = buf[row, sl] * av

        def scale_lane_range(base_row, a16, j0):
            for j in range(j0, _LANES):
                av = jnp.full((_LANES,), a16[j], jnp.float32)
                row = base_row + j
                for k in range(nvec):
                    sl = pl.ds(k * _LANES, _LANES)
                    buf[row, sl] = buf[row, sl] * av

        def group(g, ecarry):
            a16 = adj_all[ci, pl.ds(g * _LANES, _LANES)]
            scale_rows(g * _LANES, a16, _LANES)
            return ecarry

        lax.fori_loop(0, _CHUNK // _LANES, group, 0)
        rem = _CHUNK % _LANES
        if rem:
            a16 = adj_all[ci, pl.ds(_CHUNK - _LANES, _LANES)]
            scale_lane_range(_CHUNK - _LANES, a16, _LANES - rem)

    def pass_body(p, carry):
        row0 = chunk_base + p * _PASS_CHUNKS
        pltpu.sync_copy(src_hbm.at[pl.ds(row0, _PASS_CHUNKS)], src_all)
        pltpu.sync_copy(dst_hbm.at[pl.ds(row0, _PASS_CHUNKS)], dst_all)
        pltpu.sync_copy(adj_hbm.at[pl.ds(row0, _PASS_CHUNKS)], adj_all)
        def chunk_body(ci, carry2):
            pltpu.async_copy(x_hbm.at[src_all.at[ci]], bufs[0], gsems[0])
            wait_gather(bufs[0], gsems[0])
            compute_scale(bufs[0], ci)
            pltpu.sync_copy(bufs[0], acc.at[dst_all.at[ci]], add=True)
            return carry2

        lax.fori_loop(0, _PASS_CHUNKS, chunk_body, 0)
        return carry

    lax.fori_loop(0, n_passes, pass_body, 0)
    plsc.subcore_barrier()

    r0 = s * rows_per_tile
    pltpu.sync_copy(acc.at[pl.ds(r0, rows_per_tile)],
                    partial_hbm.at[c, pl.ds(r0, rows_per_tile)])


@jax.jit
def _sc_spmm(x, src2, dst2, adj2):
    n, d = x.shape
    n_chunks_total = src2.shape[0]
    nw = _NC * _NS
    per_worker_chunks = n_chunks_total // nw
    n_passes = per_worker_chunks // _PASS_CHUNKS
    n_pad = _n_pad_rows(n)
    rows_per_tile = n_pad // _NS

    mesh = plsc.VectorSubcoreMesh(core_axis_name="c", subcore_axis_name="s")
    body = functools.partial(_sc_body, n_passes, rows_per_tile, d // _LANES)

    def wrapped(x_hbm, src_hbm, dst_hbm, adj_hbm, partial_hbm,
                src_all, dst_all, adj_all, *rest):
        bufs = rest[:_NBUF]
        acc = rest[_NBUF]
        gsems = rest[_NBUF + 1:2 * _NBUF + 1]
        ssems = rest[2 * _NBUF + 1:]
        body(x_hbm, src_hbm, dst_hbm, adj_hbm, partial_hbm,
             src_all, dst_all, adj_all, bufs, acc, gsems, ssems)

    f = pl.kernel(
        wrapped,
        out_type=jax.ShapeDtypeStruct((_NC, n_pad, d), jnp.float32),
        mesh=mesh,
        scratch_types=[
            pltpu.VMEM((_PASS_CHUNKS, _CHUNK), jnp.int32),
            pltpu.VMEM((_PASS_CHUNKS, _CHUNK), jnp.int32),
            pltpu.VMEM((_PASS_CHUNKS, _CHUNK), jnp.float32),
        ] + [pltpu.VMEM((_CHUNK, d), jnp.float32)] * _NBUF + [
            pltpu.VMEM_SHARED((n_pad, d), jnp.float32),
        ] + [pltpu.SemaphoreType.DMA] * (2 * _NBUF),
    )
    return f(x, src2, dst2, adj2)


def _combine_body(p_ref, w_ref, o_ref):
    o_ref[...] = (p_ref[0] + p_ref[1]) * w_ref[...]


def _combine(partial, w2d, n):
    _, n_pad, d = partial.shape
    blk = 1000 if n % 1000 == 0 else n
    grid_r = n // blk
    return pl.pallas_call(
        _combine_body,
        grid=(grid_r,),
        in_specs=[
            pl.BlockSpec((_NC, blk, d), lambda i: (0, i, 0)),
            pl.BlockSpec((1, d), lambda i: (0, 0)),
        ],
        out_specs=pl.BlockSpec((blk, d), lambda i: (i, 0)),
        out_shape=jax.ShapeDtypeStruct((n, d), jnp.float32),
    )(partial, w2d)


def kernel(x, edge_index, adj_values, W):
    n, d = x.shape
    e = adj_values.shape[0]
    unit = _CHUNK * _NC * _NS * 8
    e_pad = ((e + unit - 1) // unit) * unit
    pad = e_pad - e
    n_pad = _n_pad_rows(n)
    spread = max(n_pad - n, 1)
    dst_fill = n + (jnp.arange(pad, dtype=jnp.int32) % spread) if pad else None
    dst = jnp.concatenate([edge_index[0], dst_fill]) if pad else edge_index[0]
    src = jnp.pad(edge_index[1], (0, pad))
    adj2 = jnp.pad(adj_values, (0, pad))
    shape2 = (e_pad // _CHUNK, _CHUNK)
    partial = _sc_spmm(x, src.reshape(shape2), dst.reshape(shape2),
                       adj2.reshape(shape2))
    return _combine(partial, W.reshape(1, d), n)


# speedup vs baseline: 11.5603x; 11.5603x over previous
import functools

import jax
import jax.numpy as jnp
from jax import lax
from jax.experimental import pallas as pl
from jax.experimental.pallas import tpu as pltpu
from jax.experimental.pallas import tpu_sc as plsc

_NC = 2
_NS = 16
_LANES = 16
_CHUNK = 80
_PASS_CHUNKS = 16
_NBUF = 1
_AHEAD = 1


def _n_pad_rows(n):
    align = _NS * 8
    return ((n + align - 1) // align) * align


def _sc_body(n_passes, rows_per_tile, nvec,
             x_hbm, src_hbm, dst_hbm, adj_hbm, partial_hbm,
             src_all, dst_all, adj_all, bufs, acc, gsems, ssems):
    c = lax.axis_index("c")
    s = lax.axis_index("s")
    wid = c * _NS + s

    zero = jnp.zeros((_LANES,), jnp.float32)
    for r in range(8):
        for k in range(nvec):
            bufs[0][r, pl.ds(k * _LANES, _LANES)] = zero

    def zcopy(b, carry):
        pltpu.sync_copy(bufs[0].at[pl.ds(0, 8)],
                        acc.at[pl.ds(s * rows_per_tile + b * 8, 8)])
        return carry

    lax.fori_loop(0, rows_per_tile // 8, zcopy, 0)
    plsc.subcore_barrier()

    chunk_base = wid * (n_passes * _PASS_CHUNKS)

    def wait_gather(buf, sem):
        pltpu.make_async_copy(x_hbm.at[pl.ds(0, _CHUNK)], buf, sem).wait()

    def wait_scatter(buf, idx_row, sem):
        pltpu.make_async_copy(
            buf, acc.at[dst_all.at[pl.ds(idx_row * _CHUNK, _CHUNK)]],
            sem).wait()

    def compute_scale(buf, ci):
        def scale_rows(base_row, a16, count):
            for j in range(count):
                av = jnp.full((_LANES,), a16[j], jnp.float32)
                row = base_row + j
                for k in range(nvec):
                    sl = pl.ds(k * _LANES, _LANES)
                    buf[row, sl] = buf[row, sl] * av

        def scale_lane_range(base_row, a16, j0):
            for j in range(j0, _LANES):
                av = jnp.full((_LANES,), a16[j], jnp.float32)
                row = base_row + j
                for k in range(nvec):
                    sl = pl.ds(k * _LANES, _LANES)
                    buf[row, sl] = buf[row, sl] * av

        def group(g, ecarry):
            a16 = adj_all[pl.ds(ci * _CHUNK + g * _LANES, _LANES)]
            scale_rows(g * _LANES, a16, _LANES)
            return ecarry

        lax.fori_loop(0, _CHUNK // _LANES, group, 0)
        rem = _CHUNK % _LANES
        if rem:
            a16 = adj_all[pl.ds(ci * _CHUNK + _CHUNK - _LANES, _LANES)]
            scale_lane_range(_CHUNK - _LANES, a16, _LANES - rem)

    def pass_body(p, carry):
        e0 = (chunk_base + p * _PASS_CHUNKS) * _CHUNK
        npass = _PASS_CHUNKS * _CHUNK
        pltpu.sync_copy(src_hbm.at[pl.ds(e0, npass)], src_all)
        pltpu.sync_copy(dst_hbm.at[pl.ds(e0, npass)], dst_all)
        pltpu.sync_copy(adj_hbm.at[pl.ds(e0, npass)], adj_all)
        def chunk_body(ci, carry2):
            pltpu.async_copy(x_hbm.at[src_all.at[pl.ds(ci * _CHUNK, _CHUNK)]],
                             bufs[0], gsems[0])
            wait_gather(bufs[0], gsems[0])
            compute_scale(bufs[0], ci)
            pltpu.sync_copy(bufs[0],
                            acc.at[dst_all.at[pl.ds(ci * _CHUNK, _CHUNK)]],
                            add=True)
            return carry2

        lax.fori_loop(0, _PASS_CHUNKS, chunk_body, 0)
        return carry

    lax.fori_loop(0, n_passes, pass_body, 0)
    plsc.subcore_barrier()

    r0 = s * rows_per_tile
    pltpu.sync_copy(acc.at[pl.ds(r0, rows_per_tile)],
                    partial_hbm.at[c, pl.ds(r0, rows_per_tile)])


@jax.jit
def _sc_spmm(x, src2, dst2, adj2):
    n, d = x.shape
    n_chunks_total = src2.shape[0] // _CHUNK
    nw = _NC * _NS
    per_worker_chunks = n_chunks_total // nw
    n_passes = per_worker_chunks // _PASS_CHUNKS
    n_pad = _n_pad_rows(n)
    rows_per_tile = n_pad // _NS

    mesh = plsc.VectorSubcoreMesh(core_axis_name="c", subcore_axis_name="s")
    body = functools.partial(_sc_body, n_passes, rows_per_tile, d // _LANES)

    def wrapped(x_hbm, src_hbm, dst_hbm, adj_hbm, partial_hbm,
                src_all, dst_all, adj_all, *rest):
        bufs = rest[:_NBUF]
        acc = rest[_NBUF]
        gsems = rest[_NBUF + 1:2 * _NBUF + 1]
        ssems = rest[2 * _NBUF + 1:]
        body(x_hbm, src_hbm, dst_hbm, adj_hbm, partial_hbm,
             src_all, dst_all, adj_all, bufs, acc, gsems, ssems)

    f = pl.kernel(
        wrapped,
        out_type=jax.ShapeDtypeStruct((_NC, n_pad, d), jnp.float32),
        mesh=mesh,
        scratch_types=[
            pltpu.VMEM((_PASS_CHUNKS * _CHUNK,), jnp.int32),
            pltpu.VMEM((_PASS_CHUNKS * _CHUNK,), jnp.int32),
            pltpu.VMEM((_PASS_CHUNKS * _CHUNK,), jnp.float32),
        ] + [pltpu.VMEM((_CHUNK, d), jnp.float32)] * _NBUF + [
            pltpu.VMEM_SHARED((n_pad, d), jnp.float32),
        ] + [pltpu.SemaphoreType.DMA] * (2 * _NBUF),
    )
    return f(x, src2, dst2, adj2)


def _combine_body(p_ref, w_ref, o_ref):
    o_ref[...] = (p_ref[0] + p_ref[1]) * w_ref[...]


def _combine(partial, w2d, n):
    _, n_pad, d = partial.shape
    blk = 1000 if n % 1000 == 0 else n
    grid_r = n // blk
    return pl.pallas_call(
        _combine_body,
        grid=(grid_r,),
        in_specs=[
            pl.BlockSpec((_NC, blk, d), lambda i: (0, i, 0)),
            pl.BlockSpec((1, d), lambda i: (0, 0)),
        ],
        out_specs=pl.BlockSpec((blk, d), lambda i: (i, 0)),
        out_shape=jax.ShapeDtypeStruct((n, d), jnp.float32),
    )(partial, w2d)


def kernel(x, edge_index, adj_values, W):
    n, d = x.shape
    e = adj_values.shape[0]
    unit = _CHUNK * _NC * _NS * 8
    e_pad = ((e + unit - 1) // unit) * unit
    pad = e_pad - e
    n_pad = _n_pad_rows(n)
    spread = max(n_pad - n, 1)
    dst_fill = n + (jnp.arange(pad, dtype=jnp.int32) % spread) if pad else None
    dst = jnp.concatenate([edge_index[0], dst_fill]) if pad else edge_index[0]
    src = jnp.pad(edge_index[1], (0, pad))
    adj2 = jnp.pad(adj_values, (0, pad))
    shape2 = (e_pad // _CHUNK, _CHUNK)
    partial = _sc_spmm(x, src.reshape(shape2), dst.reshape(shape2),
                       adj2.reshape(shape2))
    return _combine(partial, W.reshape(1, d), n)
